# baseline (speedup 1.0000x reference)
import functools

import jax
import jax.numpy as jnp
from jax import lax
from jax.experimental import pallas as pl
from jax.experimental.pallas import tpu as pltpu, tpu_sc as plsc

N_EMBD = 1024
N_EXPERTS = 16
MOE_LOSS_COEFF = 0.01

TILE = 2048

try:
    _info = plsc.get_sparse_core_info()
    NC, NS, L = _info.num_cores, _info.num_subcores, _info.num_lanes
except Exception:
    NC, NS, L = 2, 16, 16
NW = NC * NS


def _logits_body(x_ref, w_ref, lt_ref):
    lt_ref[...] = jax.lax.dot_general(
        w_ref[...], x_ref[...], (((1,), (1,)), ((), ())),
        preferred_element_type=jnp.float32)


def _tc_logits(xf, W):
    ntok = xf.shape[0]
    nsteps = ntok // TILE
    return pl.pallas_call(
        _logits_body,
        grid=(nsteps,),
        in_specs=[
            pl.BlockSpec((TILE, N_EMBD), lambda i: (i, 0)),
            pl.BlockSpec((N_EXPERTS, N_EMBD), lambda i: (0, 0)),
        ],
        out_specs=pl.BlockSpec((N_EXPERTS, TILE), lambda i: (0, i)),
        out_shape=jax.ShapeDtypeStruct((N_EXPERTS, ntok), jnp.float32),
        compiler_params=pltpu.CompilerParams(
            dimension_semantics=("arbitrary",),
        ),
    )(xf, W)


def _make_sc_router(ntok):
    tok_w = ntok // NW
    ngroups = tok_w // L

    mesh = plsc.VectorSubcoreMesh(core_axis_name="c", subcore_axis_name="s")

    @functools.partial(
        pl.kernel, mesh=mesh,
        out_type=[
            jax.ShapeDtypeStruct((2, ntok), jnp.float32),
            jax.ShapeDtypeStruct((2, ntok), jnp.int32),
            jax.ShapeDtypeStruct((NW, N_EXPERTS, L), jnp.float32),
            jax.ShapeDtypeStruct((NW, N_EXPERTS, L), jnp.float32),
        ],
        scratch_types=[
            pltpu.VMEM((N_EXPERTS, tok_w), jnp.float32),
            pltpu.VMEM((2, tok_w), jnp.float32),
            pltpu.VMEM((2, tok_w), jnp.int32),
            pltpu.VMEM((N_EXPERTS, L), jnp.float32),
            pltpu.VMEM((N_EXPERTS, L), jnp.float32),
        ],
    )
    def sc_route(lt_hbm, gates_hbm, idx_hbm, imp_hbm, cnt_hbm,
                 lt_v, gates_v, idx_v, imp_acc, cnt_acc):
        wid = lax.axis_index("s") * NC + lax.axis_index("c")
        base = wid * tok_w
        pltpu.sync_copy(lt_hbm.at[:, pl.ds(base, tok_w)], lt_v)

        zeros16 = jnp.zeros((L,), jnp.float32)
        for e in range(N_EXPERTS):
            imp_acc[e] = zeros16
            cnt_acc[e] = zeros16

        neg_inf = jnp.full((L,), -jnp.inf, jnp.float32)
        one = jnp.full((L,), 1.0, jnp.float32)

        for g in range(ngroups):
            sl = pl.ds(g * L, L)
            Ls = [lt_v[e, sl] for e in range(N_EXPERTS)]

            m = Ls[0]
            for e in range(1, N_EXPERTS):
                m = jnp.maximum(m, Ls[e])

            idx1 = jnp.full((L,), N_EXPERTS - 1, jnp.int32)
            for e in range(N_EXPERTS - 2, -1, -1):
                idx1 = jnp.where(Ls[e] == m, jnp.int32(e), idx1)

            es = [jnp.exp(Ls[e] - m) for e in range(N_EXPERTS)]
            ssum = es[0]
            for e in range(1, N_EXPERTS):
                ssum = ssum + es[e]
            r = one / ssum

            lms = [jnp.where(idx1 == e, neg_inf, Ls[e])
                   for e in range(N_EXPERTS)]
            l2 = lms[0]
            for e in range(1, N_EXPERTS):
                l2 = jnp.maximum(l2, lms[e])
            idx2 = jnp.full((L,), N_EXPERTS - 1, jnp.int32)
            for e in range(N_EXPERTS - 2, -1, -1):
                idx2 = jnp.where(lms[e] == l2, jnp.int32(e), idx2)

            g1 = one / (one + jnp.exp(l2 - m))
            gates_v[0, sl] = g1
            gates_v[1, sl] = one - g1
            idx_v[0, sl] = idx1
            idx_v[1, sl] = idx2

            for e in range(N_EXPERTS):
                plsc.addupdate(imp_acc.at[e], es[e] * r)
                plsc.addupdate(
                    cnt_acc.at[e],
                    jnp.where(idx1 == e, one, jnp.zeros((L,), jnp.float32)))

        pltpu.sync_copy(gates_v, gates_hbm.at[:, pl.ds(base, tok_w)])
        pltpu.sync_copy(idx_v, idx_hbm.at[:, pl.ds(base, tok_w)])
        pltpu.sync_copy(imp_acc, imp_hbm.at[wid])
        pltpu.sync_copy(cnt_acc, cnt_hbm.at[wid])

    return sc_route


def kernel(x, W):
    B, T, D = x.shape
    ntok = B * T
    xf = x.reshape(ntok, D)

    lt = _tc_logits(xf, W)
    gates, idx, imp_p, cnt_p = _make_sc_router(ntok)(lt)

    gates = gates.T.reshape(B, T, 2)
    idx = idx.T.reshape(B, T, 2)
    imp = jnp.sum(imp_p, axis=(0, 2))
    cnt = jnp.sum(cnt_p, axis=(0, 2))
    scale = MOE_LOSS_COEFF * N_EXPERTS / float(ntok * ntok)
    aux = jnp.sum(imp * cnt) * scale
    return (gates, idx, aux)


# speedup vs baseline: 2.9155x; 2.9155x over previous
import jax
import jax.numpy as jnp
from jax.experimental import pallas as pl
from jax.experimental.pallas import tpu as pltpu

N_EMBD = 1024
N_EXPERTS = 16
MOE_LOSS_COEFF = 0.01

TILE = 2048


def _router_body(x_ref, w_ref, gates_ref, idx_ref, aux_ref, imp_ref, cnt_ref):
    i = pl.program_id(0)
    nsteps = pl.num_programs(0)

    @pl.when(i == 0)
    def _init():
        imp_ref[...] = jnp.zeros_like(imp_ref)
        cnt_ref[...] = jnp.zeros_like(cnt_ref)

    lt = jax.lax.dot_general(
        w_ref[...], x_ref[...], (((1,), (1,)), ((), ())),
        preferred_element_type=jnp.float32)

    m = jnp.max(lt, axis=0, keepdims=True)
    e = jnp.exp(lt - m)
    s = jnp.sum(e, axis=0, keepdims=True)
    probs = e / s

    eidx = jax.lax.broadcasted_iota(jnp.int32, lt.shape, 0)
    idx1 = jnp.min(jnp.where(lt == m, eidx, N_EXPERTS),
                   axis=0, keepdims=True)
    hit1 = eidx == idx1
    lm = jnp.where(hit1, -jnp.inf, lt)
    l2 = jnp.max(lm, axis=0, keepdims=True)
    idx2 = jnp.min(jnp.where(lm == l2, eidx, N_EXPERTS),
                   axis=0, keepdims=True)

    g1 = 1.0 / (1.0 + jnp.exp(l2 - m))
    gates_ref[...] = jnp.concatenate([g1, 1.0 - g1], axis=0)
    idx_ref[...] = jnp.concatenate([idx1, idx2], axis=0)

    imp_ref[...] += jnp.sum(probs, axis=1, keepdims=True)
    cnt_ref[...] += jnp.sum(jnp.where(hit1, 1.0, 0.0), axis=1, keepdims=True)

    @pl.when(i == nsteps - 1)
    def _fin():
        ntok = nsteps * TILE
        scale = MOE_LOSS_COEFF * N_EXPERTS / float(ntok * ntok)
        aux_ref[...] = jnp.sum(imp_ref[...] * cnt_ref[...],
                               keepdims=True) * scale


def kernel(x, W):
    B, T, D = x.shape
    ntok = B * T
    xf = x.reshape(ntok, D)
    nsteps = ntok // TILE

    gates, idx, aux = pl.pallas_call(
        _router_body,
        grid=(nsteps,),
        in_specs=[
            pl.BlockSpec((TILE, D), lambda i: (i, 0)),
            pl.BlockSpec((N_EXPERTS, D), lambda i: (0, 0)),
        ],
        out_specs=[
            pl.BlockSpec((2, TILE), lambda i: (0, i)),
            pl.BlockSpec((2, TILE), lambda i: (0, i)),
            pl.BlockSpec((1, 1), lambda i: (0, 0)),
        ],
        out_shape=[
            jax.ShapeDtypeStruct((2, ntok), jnp.float32),
            jax.ShapeDtypeStruct((2, ntok), jnp.int32),
            jax.ShapeDtypeStruct((1, 1), jnp.float32),
        ],
        scratch_shapes=[
            pltpu.VMEM((N_EXPERTS, 1), jnp.float32),
            pltpu.VMEM((N_EXPERTS, 1), jnp.float32),
        ],
        compiler_params=pltpu.CompilerParams(
            dimension_semantics=("arbitrary",),
        ),
    )(xf, W)

    gates = gates.T.reshape(B, T, 2)
    idx = idx.T.reshape(B, T, 2)
    return (gates, idx, aux.reshape(()))
